# baseline (speedup 1.0000x reference)
import jax
import jax.numpy as jnp
from jax import lax
from jax.experimental import pallas as pl
from jax.experimental.pallas import tpu as pltpu
from jax.experimental.pallas import tpu_sc as plsc

_NC = 2
_NS = 16
_NW = _NC * _NS
_B, _C, _H, _W = 8, 96, 224, 224
_IMGS = _B * _C
_PER_W = _IMGS // _NW
_SPLIT = 4
_CR = _H // _SPLIT
_STEPS = _PER_W * _SPLIT
_NG = 3
_NO = 3
_LOOKAHEAD = 3
_SL = _W // 16
_GROUP = 12

_mesh = plsc.VectorSubcoreMesh(
    core_axis_name="c", subcore_axis_name="s", num_cores=_NC, num_subcores=_NS
)


def _body(x4_hbm, pmap_hbm, out_hbm, pv, tab, *rest):
    gbufs = rest[:_NG]
    obufs = rest[_NG:_NG + _NO]
    sem_g = rest[_NG + _NO:2 * _NG + _NO]
    sem_o = rest[2 * _NG + _NO:2 * _NG + 2 * _NO]

    wid = lax.axis_index("s") * _NC + lax.axis_index("c")
    m = jnp.bitwise_and(wid, 3)
    bq = lax.shift_right_logical(wid, 2)
    pltpu.sync_copy(pmap_hbm.at[m], pv)
    for j in range(_PER_W):
        v = pv[j // 16]
        tab[j] = v[j % 16]

    def fire_gather(jj, q, sg):
        sim = bq * _C + tab[jj]
        pltpu.async_copy(
            x4_hbm.at[pl.ds(0, 2), sim, pl.ds(q * _CR, _CR)],
            gbufs[sg], sem_g[sg])

    for k in range(_LOOKAHEAD):
        fire_gather(jnp.int32(0), k % _SPLIT, k % _NG)

    def group_step(g, _):
        k0 = g * _GROUP
        for u in range(_GROUP):
            k = k0 + u
            sg, so, q = u % _NG, u % _NO, u % _SPLIT
            j = g * (_GROUP // _SPLIT) + u // _SPLIT
            oim = wid * _PER_W + j

            @pl.when(k >= _NO)
            def _reclaim():
                pltpu.make_async_copy(
                    obufs[so], out_hbm.at[oim, pl.ds(q * _CR, _CR)],
                    sem_o[so]).wait()

            pltpu.make_async_copy(
                x4_hbm.at[pl.ds(0, 2), 0, pl.ds(0, _CR)],
                gbufs[sg], sem_g[sg]).wait()

            gb, ob = gbufs[sg], obufs[so]

            @plsc.parallel_loop(0, _CR)
            def _add(r):
                for t in range(_SL):
                    o = t * 16
                    ob[r, pl.ds(o, 16)] = (
                        gb[0, r, pl.ds(o, 16)] + gb[1, r, pl.ds(o, 16)])

            pltpu.async_copy(
                ob, out_hbm.at[oim, pl.ds(q * _CR, _CR)], sem_o[so])

            u2 = u + _LOOKAHEAD
            j2 = g * (_GROUP // _SPLIT) + u2 // _SPLIT

            @pl.when(k + _LOOKAHEAD < _STEPS)
            def _prefetch():
                fire_gather(j2, u2 % _SPLIT, u2 % _NG)

        return ()

    lax.fori_loop(0, _STEPS // _GROUP, group_step, ())

    for so in range(_NO):
        pltpu.make_async_copy(
            obufs[so], out_hbm.at[0, pl.ds(0, _CR)], sem_o[so]).wait()


_sc_add_shuffle = pl.kernel(
    _body,
    out_type=jax.ShapeDtypeStruct((_IMGS, _H, _W), jnp.float32),
    mesh=_mesh,
    scratch_types=[
        pltpu.VMEM((2, 16), jnp.int32),
        pltpu.SMEM((32,), jnp.int32),
    ]
    + [pltpu.VMEM((2, _CR, _W), jnp.float32) for _ in range(_NG)]
    + [pltpu.VMEM((_CR, _W), jnp.float32) for _ in range(_NO)]
    + [pltpu.SemaphoreType.DMA for _ in range(_NG + _NO)],
)


def kernel(x):
    perm = jax.random.permutation(jax.random.key(42), _C).astype(jnp.int32)
    pmap = jnp.pad(perm.reshape(4, 24), ((0, 0), (0, 8))).reshape(4, 2, 16)

    x4 = x.reshape(2, _IMGS, _H, _W)
    out = _sc_add_shuffle(x4, pmap)
    return out.reshape(_B, _C, _H, _W)
